# baseline (speedup 1.0000x reference)
import jax
import jax.numpy as jnp
from jax.experimental import pallas as pl

_BN = 8192
_SUB = 8192


def _obrq_kernel(x_ref, R_ref, c_ref, out_ref):
    R = R_ref[...]
    c = c_ref[...]
    Rb = R.astype(jnp.bfloat16)
    cn = jnp.sum(c * c, axis=-1, keepdims=True)
    crot = jax.lax.dot_general(
        c, R, (((1,), (1,)), ((), ())),
        preferred_element_type=jnp.float32)

    nt = _BN // _SUB
    xs = [x_ref[t * _SUB:(t + 1) * _SUB, :] for t in range(nt)]
    norms = [jnp.sqrt(jnp.sum(x * x, axis=-1, keepdims=True)) for x in xs]
    xns = [x * (1.0 / (n_ + 1e-8)) for x, n_ in zip(xs, norms)]
    xrs = [jax.lax.dot_general(xn, R, (((1,), (0,)), ((), ())),
                               preferred_element_type=jnp.float32)
           for xn in xns]
    ms = [jax.lax.dot_general(xr, c, (((1,), (1,)), ((), ())),
                              preferred_element_type=jnp.float32)
          for xr in xrs]
    sels = [(cn[1, 0] - 2.0 * m[:, 1:2]) < (cn[0, 0] - 2.0 * m[:, 0:1])
            for m in ms]
    x_mses = [jnp.where(s, c[1:2, :], c[0:1, :]) for s in sels]
    residuals = [xr - xm for xr, xm in zip(xrs, x_mses)]
    resbs = [r.astype(jnp.bfloat16) for r in residuals]
    rbits = [jax.lax.bitcast_convert_type(rb, jnp.uint16) for rb in resbs]
    signss = [jax.lax.bitcast_convert_type(
                  (b & jnp.uint16(0x8000)) | jnp.uint16(0x3F80),
                  jnp.bfloat16)
              for b in rbits]
    ones_col = jnp.full((256, 1), 1.0 / 256.0, dtype=jnp.bfloat16)
    absrs = [jax.lax.bitcast_convert_type(b & jnp.uint16(0x7FFF), jnp.bfloat16)
             for b in rbits]
    scales = [jax.lax.dot_general(ar, ones_col,
                                  (((1,), (0,)), ((), ())),
                                  preferred_element_type=jnp.float32)
              for ar in absrs]
    srots = [jax.lax.dot_general(sg, Rb, (((1,), (1,)), ((), ())),
                                 preferred_element_type=jnp.float32)
             for sg in signss]
    for t in range(nt):
        x_mse_rot = jnp.where(sels[t], crot[1:2, :], crot[0:1, :])
        recon = x_mse_rot + scales[t] * srots[t]
        out_ref[t * _SUB:(t + 1) * _SUB, :] = recon * norms[t]


@jax.jit
def kernel(x, R, centroids):
    n, d = x.shape
    grid = (n // _BN,)
    return pl.pallas_call(
        _obrq_kernel,
        grid=grid,
        in_specs=[
            pl.BlockSpec((_BN, d), lambda i: (i, 0)),
            pl.BlockSpec((d, d), lambda i: (0, 0)),
            pl.BlockSpec(centroids.shape, lambda i: (0, 0)),
        ],
        out_specs=pl.BlockSpec((_BN, d), lambda i: (i, 0)),
        out_shape=jax.ShapeDtypeStruct((n, d), jnp.float32),
    )(x, R, centroids)


# speedup vs baseline: 1.0157x; 1.0157x over previous
import jax
import jax.numpy as jnp
from jax.experimental import pallas as pl

_BN = 4096


def _obrq_kernel(x_ref, R_ref, c_ref, out_ref):
    x = x_ref[...]
    R = R_ref[...]
    c = c_ref[...]
    d = R.shape[0]
    Rb = R.astype(jnp.bfloat16)
    cn = jnp.sum(c * c, axis=-1, keepdims=True)
    crot = jax.lax.dot_general(
        c, R, (((1,), (1,)), ((), ())),
        preferred_element_type=jnp.float32)

    norm = jnp.sqrt(jnp.sum(x * x, axis=-1, keepdims=True))
    xn = x * (1.0 / (norm + 1e-8))
    xr = jax.lax.dot_general(
        xn, R, (((1,), (0,)), ((), ())),
        preferred_element_type=jnp.float32)
    m = jax.lax.dot_general(
        xr, c, (((1,), (1,)), ((), ())),
        preferred_element_type=jnp.float32)

    sel1 = (cn[1, 0] - 2.0 * m[:, 1:2]) < (cn[0, 0] - 2.0 * m[:, 0:1])
    x_mse = jnp.where(sel1, c[1:2, :], c[0:1, :])
    residual = xr - x_mse

    rbits = jax.lax.bitcast_convert_type(
        residual.astype(jnp.bfloat16), jnp.uint16)
    signs = jax.lax.bitcast_convert_type(
        (rbits & jnp.uint16(0x8000)) | jnp.uint16(0x3F80), jnp.bfloat16)
    absr = jax.lax.bitcast_convert_type(
        rbits & jnp.uint16(0x7FFF), jnp.bfloat16)
    ones_col = jnp.full((d, 1), 1.0 / d, dtype=jnp.bfloat16)
    scale = jax.lax.dot_general(
        absr, ones_col, (((1,), (0,)), ((), ())),
        preferred_element_type=jnp.float32)

    srot = jax.lax.dot_general(
        signs, Rb, (((1,), (1,)), ((), ())),
        preferred_element_type=jnp.float32)
    x_mse_rot = jnp.where(sel1, crot[1:2, :], crot[0:1, :])
    recon = x_mse_rot + scale * srot
    out_ref[...] = recon * norm


@jax.jit
def kernel(x, R, centroids):
    n, d = x.shape
    grid = (n // _BN,)
    return pl.pallas_call(
        _obrq_kernel,
        grid=grid,
        in_specs=[
            pl.BlockSpec((_BN, d), lambda i: (i, 0)),
            pl.BlockSpec((d, d), lambda i: (0, 0)),
            pl.BlockSpec(centroids.shape, lambda i: (0, 0)),
        ],
        out_specs=pl.BlockSpec((_BN, d), lambda i: (i, 0)),
        out_shape=jax.ShapeDtypeStruct((n, d), jnp.float32),
    )(x, R, centroids)
